# baseline (speedup 1.0000x reference)
import jax
import jax.numpy as jnp
from jax import lax
from jax.experimental import pallas as pl
from jax.experimental.pallas import tpu as pltpu
from jax.experimental.pallas import tpu_sc as plsc

_NC = 2
_NS = 16
_W = 80


def _dot_t(x, w):
    return lax.dot_general(x, w, (((1,), (1,)), ((), ())),
                           preferred_element_type=jnp.float32)


def _mm_split_body(x_ref, wl_ref, wr_ref, b_ref, ol_ref, or_ref):
    x = x_ref[...]
    ol_ref[...] = _dot_t(x, wl_ref[...])
    or_ref[...] = _dot_t(x, wr_ref[...]) + b_ref[...]


def _mm_split(x, wl, wr, b2d, grid=1):
    n = x.shape[0]
    d = x.shape[1]
    k = wl.shape[0]
    bn = n // grid
    return pl.pallas_call(
        _mm_split_body,
        grid=(grid,),
        in_specs=[pl.BlockSpec((bn, d), lambda i: (i, 0)),
                  pl.BlockSpec(wl.shape, lambda i: (0, 0)),
                  pl.BlockSpec(wr.shape, lambda i: (0, 0)),
                  pl.BlockSpec(b2d.shape, lambda i: (0, 0))],
        out_specs=[pl.BlockSpec((bn, k), lambda i: (i, 0)),
                   pl.BlockSpec((bn, k), lambda i: (i, 0))],
        out_shape=[jax.ShapeDtypeStruct((n, k), jnp.float32),
                   jax.ShapeDtypeStruct((n, k), jnp.float32)],
    )(x, wl, wr, b2d)


def _layer2_body(aggs_ref, wl_ref, wr_ref, b_ref, o_ref):
    c = wl_ref.shape[0]
    h = jnp.maximum(aggs_ref[0] + aggs_ref[1], 0.0)
    o_ref[:, :c] = _dot_t(h, wl_ref[...])
    o_ref[:, c:] = _dot_t(h, wr_ref[...]) + b_ref[...]


def _layer2(aggs, wl, wr, b2d, grid=1):
    n = aggs.shape[1]
    hd = aggs.shape[2]
    c = wl.shape[0]
    bn = n // grid
    return pl.pallas_call(
        _layer2_body,
        grid=(grid,),
        in_specs=[pl.BlockSpec((2, bn, hd), lambda i: (0, i, 0)),
                  pl.BlockSpec(wl.shape, lambda i: (0, 0)),
                  pl.BlockSpec(wr.shape, lambda i: (0, 0)),
                  pl.BlockSpec(b2d.shape, lambda i: (0, 0))],
        out_specs=pl.BlockSpec((bn, 2 * c), lambda i: (i, 0)),
        out_shape=jax.ShapeDtypeStruct((n, 2 * c), jnp.float32),
    )(aggs, wl, wr, b2d)


def _combine_body(aggs_ref, o_ref):
    o_ref[...] = aggs_ref[0] + aggs_ref[1]


def _combine(aggs2w):
    return pl.pallas_call(
        _combine_body,
        out_shape=jax.ShapeDtypeStruct(aggs2w.shape[1:], jnp.float32),
    )(aggs2w)


def _seg_sum_sc(feat, ei, init, init_col, nb, g, w=_W):
    f = feat.shape[1]
    n = init.shape[0]
    e = ei.shape[1]
    nw = _NC * _NS
    epw = e // nw
    cpw = epw // w
    rpt = n // _NS

    mesh = plsc.VectorSubcoreMesh(
        core_axis_name="c", subcore_axis_name="s",
        num_cores=_NC, num_subcores=_NS)

    def body(feat_hbm, ei_hbm, init_hbm, out_hbm,
             acc, sidx, didx, rbufs, gsems, ssems):
        c = lax.axis_index("c")
        s = lax.axis_index("s")
        wid = s * _NC + c
        r0 = s * rpt
        e0 = pl.multiple_of(wid * epw, 8)

        pltpu.async_copy(ei_hbm.at[0, pl.ds(e0, epw)], sidx, gsems[0])
        pltpu.async_copy(ei_hbm.at[1, pl.ds(e0, epw)], didx, gsems[1])

        @pl.when(c == 0)
        def _():
            pltpu.sync_copy(
                init_hbm.at[pl.ds(r0, rpt), pl.ds(init_col, f)],
                acc.at[pl.ds(r0, rpt)])

        @pl.when(c == 1)
        def _():
            zv = jnp.zeros((16,), jnp.float32)

            def zb(r, carry):
                for q in range(f // 16):
                    rbufs[0][r, pl.ds(q * 16, 16)] = zv
                return carry

            lax.fori_loop(0, w, zb, 0)
            nfull = rpt // w
            for t in range(nfull):
                pltpu.sync_copy(rbufs[0], acc.at[pl.ds(r0 + t * w, w)])
            rem = rpt - nfull * w
            if rem:
                pltpu.sync_copy(rbufs[0].at[pl.ds(0, rem)],
                                acc.at[pl.ds(r0 + nfull * w, rem)])

        pltpu.make_async_copy(ei_hbm.at[0, pl.ds(e0, epw)], sidx,
                              gsems[0]).wait()
        pltpu.make_async_copy(ei_hbm.at[1, pl.ds(e0, epw)], didx,
                              gsems[1]).wait()

        def cidx(k):
            return pl.multiple_of(k * w, 8)

        def gather(k, b):
            pltpu.async_copy(feat_hbm.at[sidx.at[pl.ds(cidx(k), w)]],
                             rbufs[b], gsems[b])

        def gwait(b):
            pltpu.make_async_copy(feat_hbm.at[sidx.at[pl.ds(0, w)]],
                                  rbufs[b], gsems[b]).wait()

        def scat(k, b):
            pltpu.async_copy(rbufs[b], acc.at[didx.at[pl.ds(cidx(k), w)]],
                             ssems[b], add=True)

        def swait(b):
            pltpu.make_async_copy(rbufs[b], acc.at[didx.at[pl.ds(0, w)]],
                                  ssems[b]).wait()

        for k in range(g):
            gather(k, k % nb)
        plsc.subcore_barrier()

        def chunk_step(k, b):
            gwait(b)
            scat(k, b)
            b2 = (b + g) % nb

            @pl.when(k + g < cpw)
            def _():
                @pl.when(k >= nb - g)
                def _():
                    swait(b2)

                gather(k + g, b2)

        def loop_body(i, carry):
            for b in range(nb):
                chunk_step(i * nb + b, b)
            return carry

        nloop = cpw // nb
        lax.fori_loop(0, nloop, loop_body, 0)
        for k in range(nloop * nb, cpw):
            chunk_step(k, k % nb)
        for b in range(nb):
            swait(b)

        plsc.subcore_barrier()
        pltpu.sync_copy(acc.at[pl.ds(r0, rpt)], out_hbm.at[c, pl.ds(r0, rpt)])

    kern = pl.kernel(
        body,
        out_type=jax.ShapeDtypeStruct((_NC, n, f), jnp.float32),
        mesh=mesh,
        scratch_types=[
            pltpu.VMEM_SHARED((n, f), jnp.float32),
            pltpu.VMEM((epw,), jnp.int32),
            pltpu.VMEM((epw,), jnp.int32),
            [pltpu.VMEM((w, f), jnp.float32) for _ in range(nb)],
            [pltpu.SemaphoreType.DMA for _ in range(nb)],
            [pltpu.SemaphoreType.DMA for _ in range(nb)],
        ],
        compiler_params=pltpu.CompilerParams(use_tc_tiling_on_sc=False),
    )
    return kern(feat, ei, init)


def kernel(x, edge_index, W1l, b1, W1r, W2l, b2, W2r):
    n, d = x.shape
    h = W1l.shape[0]
    c = W2l.shape[0]

    xl, xr = _mm_split(x, W1l, W1r, b1[None, :])

    aggs1 = _seg_sum_sc(xl, edge_index, xr, 0, nb=6, g=4, w=40)

    y2 = _layer2(aggs1, W2l, W2r, b2[None, :])

    aggs2 = _seg_sum_sc(y2[:, :c], edge_index, y2, c, nb=12, g=8, w=40)

    out_p = _combine(aggs2.reshape(2, n // 2, 2 * c))
    return out_p.reshape(n, c)


# speedup vs baseline: 1.0272x; 1.0272x over previous
import jax
import jax.numpy as jnp
from jax import lax
from jax.experimental import pallas as pl
from jax.experimental.pallas import tpu as pltpu
from jax.experimental.pallas import tpu_sc as plsc

_NC = 2
_NS = 16
_W = 80


def _dot_t(x, w):
    return lax.dot_general(x, w, (((1,), (1,)), ((), ())),
                           preferred_element_type=jnp.float32)


def _mm_split_body(x_ref, wl_ref, wr_ref, b_ref, ol_ref, or_ref):
    x = x_ref[...]
    ol_ref[...] = _dot_t(x, wl_ref[...])
    or_ref[...] = _dot_t(x, wr_ref[...]) + b_ref[...]


def _mm_split(x, wl, wr, b2d, grid=1):
    n = x.shape[0]
    d = x.shape[1]
    k = wl.shape[0]
    bn = n // grid
    return pl.pallas_call(
        _mm_split_body,
        grid=(grid,),
        in_specs=[pl.BlockSpec((bn, d), lambda i: (i, 0)),
                  pl.BlockSpec(wl.shape, lambda i: (0, 0)),
                  pl.BlockSpec(wr.shape, lambda i: (0, 0)),
                  pl.BlockSpec(b2d.shape, lambda i: (0, 0))],
        out_specs=[pl.BlockSpec((bn, k), lambda i: (i, 0)),
                   pl.BlockSpec((bn, k), lambda i: (i, 0))],
        out_shape=[jax.ShapeDtypeStruct((n, k), jnp.float32),
                   jax.ShapeDtypeStruct((n, k), jnp.float32)],
    )(x, wl, wr, b2d)


def _layer2_body(aggs_ref, wl_ref, wr_ref, b_ref, o_ref):
    c = wl_ref.shape[0]
    h = jnp.maximum(aggs_ref[0] + aggs_ref[1], 0.0)
    o_ref[:, :c] = _dot_t(h, wl_ref[...])
    o_ref[:, c:] = _dot_t(h, wr_ref[...]) + b_ref[...]


def _layer2(aggs, wl, wr, b2d, grid=1):
    n = aggs.shape[1]
    hd = aggs.shape[2]
    c = wl.shape[0]
    bn = n // grid
    return pl.pallas_call(
        _layer2_body,
        grid=(grid,),
        in_specs=[pl.BlockSpec((2, bn, hd), lambda i: (0, i, 0)),
                  pl.BlockSpec(wl.shape, lambda i: (0, 0)),
                  pl.BlockSpec(wr.shape, lambda i: (0, 0)),
                  pl.BlockSpec(b2d.shape, lambda i: (0, 0))],
        out_specs=pl.BlockSpec((bn, 2 * c), lambda i: (i, 0)),
        out_shape=jax.ShapeDtypeStruct((n, 2 * c), jnp.float32),
    )(aggs, wl, wr, b2d)


def _combine_body(aggs_ref, o_ref):
    o_ref[...] = aggs_ref[0] + aggs_ref[1]


def _combine(aggs2w):
    return pl.pallas_call(
        _combine_body,
        out_shape=jax.ShapeDtypeStruct(aggs2w.shape[1:], jnp.float32),
    )(aggs2w)


def _seg_sum_sc(feat, ei, init, init_col, nb, g, w=_W):
    f = feat.shape[1]
    n = init.shape[0]
    e = ei.shape[1]
    nw = _NC * _NS
    epw = e // nw
    cpw = epw // w
    rpt = n // _NS

    mesh = plsc.VectorSubcoreMesh(
        core_axis_name="c", subcore_axis_name="s",
        num_cores=_NC, num_subcores=_NS)

    def body(feat_hbm, ei_hbm, init_hbm, out_hbm,
             acc, sidx, didx, rbufs, gsems, ssems):
        c = lax.axis_index("c")
        s = lax.axis_index("s")
        wid = s * _NC + c
        r0 = s * rpt
        e0 = pl.multiple_of(wid * epw, 8)

        pltpu.async_copy(ei_hbm.at[0, pl.ds(e0, epw)], sidx, gsems[0])
        pltpu.async_copy(ei_hbm.at[1, pl.ds(e0, epw)], didx, gsems[1])

        @pl.when(c == 0)
        def _():
            pltpu.sync_copy(
                init_hbm.at[pl.ds(r0, rpt), pl.ds(init_col, f)],
                acc.at[pl.ds(r0, rpt)])

        @pl.when(c == 1)
        def _():
            zv = jnp.zeros((16,), jnp.float32)

            def zb(r, carry):
                for q in range(f // 16):
                    rbufs[0][r, pl.ds(q * 16, 16)] = zv
                return carry

            lax.fori_loop(0, w, zb, 0)
            nfull = rpt // w
            for t in range(nfull):
                pltpu.sync_copy(rbufs[0], acc.at[pl.ds(r0 + t * w, w)])
            rem = rpt - nfull * w
            if rem:
                pltpu.sync_copy(rbufs[0].at[pl.ds(0, rem)],
                                acc.at[pl.ds(r0 + nfull * w, rem)])

        pltpu.make_async_copy(ei_hbm.at[0, pl.ds(e0, epw)], sidx,
                              gsems[0]).wait()
        pltpu.make_async_copy(ei_hbm.at[1, pl.ds(e0, epw)], didx,
                              gsems[1]).wait()

        def cidx(k):
            return pl.multiple_of(k * w, 8)

        def gather(k, b):
            pltpu.async_copy(feat_hbm.at[sidx.at[pl.ds(cidx(k), w)]],
                             rbufs[b], gsems[b])

        def gwait(b):
            pltpu.make_async_copy(feat_hbm.at[sidx.at[pl.ds(0, w)]],
                                  rbufs[b], gsems[b]).wait()

        def scat(k, b):
            pltpu.async_copy(rbufs[b], acc.at[didx.at[pl.ds(cidx(k), w)]],
                             ssems[b], add=True)

        def swait(b):
            pltpu.make_async_copy(rbufs[b], acc.at[didx.at[pl.ds(0, w)]],
                                  ssems[b]).wait()

        for k in range(g):
            gather(k, k % nb)
        plsc.subcore_barrier()

        def chunk_step(k, b):
            gwait(b)
            scat(k, b)
            b2 = (b + g) % nb

            @pl.when(k + g < cpw)
            def _():
                @pl.when(k >= nb - g)
                def _():
                    swait(b2)

                gather(k + g, b2)

        def loop_body(i, carry):
            for b in range(nb):
                chunk_step(i * nb + b, b)
            return carry

        nloop = cpw // nb
        lax.fori_loop(0, nloop, loop_body, 0)
        for k in range(nloop * nb, cpw):
            chunk_step(k, k % nb)
        for b in range(nb):
            swait(b)

        plsc.subcore_barrier()
        pltpu.sync_copy(acc.at[pl.ds(r0, rpt)], out_hbm.at[c, pl.ds(r0, rpt)])

    kern = pl.kernel(
        body,
        out_type=jax.ShapeDtypeStruct((_NC, n, f), jnp.float32),
        mesh=mesh,
        scratch_types=[
            pltpu.VMEM_SHARED((n, f), jnp.float32),
            pltpu.VMEM((epw,), jnp.int32),
            pltpu.VMEM((epw,), jnp.int32),
            [pltpu.VMEM((w, f), jnp.float32) for _ in range(nb)],
            [pltpu.SemaphoreType.DMA for _ in range(nb)],
            [pltpu.SemaphoreType.DMA for _ in range(nb)],
        ],
        compiler_params=pltpu.CompilerParams(use_tc_tiling_on_sc=False),
    )
    return kern(feat, ei, init)


def kernel(x, edge_index, W1l, b1, W1r, W2l, b2, W2r):
    n, d = x.shape
    h = W1l.shape[0]
    c = W2l.shape[0]

    xl, xr = _mm_split(x, W1l, W1r, b1[None, :])

    aggs1 = _seg_sum_sc(xl, edge_index, xr, 0, nb=6, g=5, w=40)

    y2 = _layer2(aggs1, W2l, W2r, b2[None, :])

    aggs2 = _seg_sum_sc(y2[:, :c], edge_index, y2, c, nb=6, g=4, w=_W)

    out_p = _combine(aggs2.reshape(2, n // 2, 2 * c))
    return out_p.reshape(n, c)


# speedup vs baseline: 1.0286x; 1.0014x over previous
import jax
import jax.numpy as jnp
from jax import lax
from jax.experimental import pallas as pl
from jax.experimental.pallas import tpu as pltpu
from jax.experimental.pallas import tpu_sc as plsc

_NC = 2
_NS = 16
_W = 80


def _dot_t(x, w):
    return lax.dot_general(x, w, (((1,), (1,)), ((), ())),
                           preferred_element_type=jnp.float32)


def _mm_split_body(x_ref, wl_ref, wr_ref, b_ref, ol_ref, or_ref):
    x = x_ref[...]
    ol_ref[...] = _dot_t(x, wl_ref[...])
    or_ref[...] = _dot_t(x, wr_ref[...]) + b_ref[...]


def _mm_split(x, wl, wr, b2d, grid=1):
    n = x.shape[0]
    d = x.shape[1]
    k = wl.shape[0]
    bn = n // grid
    return pl.pallas_call(
        _mm_split_body,
        grid=(grid,),
        in_specs=[pl.BlockSpec((bn, d), lambda i: (i, 0)),
                  pl.BlockSpec(wl.shape, lambda i: (0, 0)),
                  pl.BlockSpec(wr.shape, lambda i: (0, 0)),
                  pl.BlockSpec(b2d.shape, lambda i: (0, 0))],
        out_specs=[pl.BlockSpec((bn, k), lambda i: (i, 0)),
                   pl.BlockSpec((bn, k), lambda i: (i, 0))],
        out_shape=[jax.ShapeDtypeStruct((n, k), jnp.float32),
                   jax.ShapeDtypeStruct((n, k), jnp.float32)],
    )(x, wl, wr, b2d)


def _layer2_body(aggs_ref, wl_ref, wr_ref, b_ref, o_ref):
    c = wl_ref.shape[0]
    h = jnp.maximum(aggs_ref[0] + aggs_ref[1], 0.0)
    o_ref[:, :c] = _dot_t(h, wl_ref[...])
    o_ref[:, c:] = _dot_t(h, wr_ref[...]) + b_ref[...]


def _layer2(aggs, wl, wr, b2d, grid=1):
    n = aggs.shape[1]
    hd = aggs.shape[2]
    c = wl.shape[0]
    bn = n // grid
    return pl.pallas_call(
        _layer2_body,
        grid=(grid,),
        in_specs=[pl.BlockSpec((2, bn, hd), lambda i: (0, i, 0)),
                  pl.BlockSpec(wl.shape, lambda i: (0, 0)),
                  pl.BlockSpec(wr.shape, lambda i: (0, 0)),
                  pl.BlockSpec(b2d.shape, lambda i: (0, 0))],
        out_specs=pl.BlockSpec((bn, 2 * c), lambda i: (i, 0)),
        out_shape=jax.ShapeDtypeStruct((n, 2 * c), jnp.float32),
    )(aggs, wl, wr, b2d)


def _combine_body(aggs_ref, o_ref):
    o_ref[...] = aggs_ref[0] + aggs_ref[1]


def _combine(aggs2w):
    return pl.pallas_call(
        _combine_body,
        out_shape=jax.ShapeDtypeStruct(aggs2w.shape[1:], jnp.float32),
    )(aggs2w)


def _seg_sum_sc(feat, ei, init, init_col, nb, g, w=_W):
    f = feat.shape[1]
    n = init.shape[0]
    e = ei.shape[1]
    nw = _NC * _NS
    epw = e // nw
    cpw = epw // w
    rpt = n // _NS

    mesh = plsc.VectorSubcoreMesh(
        core_axis_name="c", subcore_axis_name="s",
        num_cores=_NC, num_subcores=_NS)

    def body(feat_hbm, ei_hbm, init_hbm, out_hbm,
             acc, sidx, didx, rbufs, gsems, ssems):
        c = lax.axis_index("c")
        s = lax.axis_index("s")
        wid = s * _NC + c
        r0 = s * rpt
        e0 = pl.multiple_of(wid * epw, 8)

        pltpu.async_copy(ei_hbm.at[0, pl.ds(e0, epw)], sidx, gsems[0])
        pltpu.async_copy(ei_hbm.at[1, pl.ds(e0, epw)], didx, gsems[1])

        @pl.when(c == 0)
        def _():
            pltpu.sync_copy(
                init_hbm.at[pl.ds(r0, rpt), pl.ds(init_col, f)],
                acc.at[pl.ds(r0, rpt)])

        @pl.when(c == 1)
        def _():
            zv = jnp.zeros((16,), jnp.float32)

            def zb(r, carry):
                for q in range(f // 16):
                    rbufs[0][r, pl.ds(q * 16, 16)] = zv
                return carry

            lax.fori_loop(0, w, zb, 0)
            nfull = rpt // w
            for t in range(nfull):
                pltpu.sync_copy(rbufs[0], acc.at[pl.ds(r0 + t * w, w)])
            rem = rpt - nfull * w
            if rem:
                pltpu.sync_copy(rbufs[0].at[pl.ds(0, rem)],
                                acc.at[pl.ds(r0 + nfull * w, rem)])

        pltpu.make_async_copy(ei_hbm.at[0, pl.ds(e0, epw)], sidx,
                              gsems[0]).wait()
        pltpu.make_async_copy(ei_hbm.at[1, pl.ds(e0, epw)], didx,
                              gsems[1]).wait()

        def cidx(k):
            return pl.multiple_of(k * w, 8)

        def gather(k, b):
            pltpu.async_copy(feat_hbm.at[sidx.at[pl.ds(cidx(k), w)]],
                             rbufs[b], gsems[b])

        def gwait(b):
            pltpu.make_async_copy(feat_hbm.at[sidx.at[pl.ds(0, w)]],
                                  rbufs[b], gsems[b]).wait()

        def scat(k, b):
            pltpu.async_copy(rbufs[b], acc.at[didx.at[pl.ds(cidx(k), w)]],
                             ssems[b], add=True)

        def swait(b):
            pltpu.make_async_copy(rbufs[b], acc.at[didx.at[pl.ds(0, w)]],
                                  ssems[b]).wait()

        for k in range(g):
            gather(k, k % nb)
        plsc.subcore_barrier()

        def chunk_step(k, b):
            gwait(b)
            scat(k, b)
            b2 = (b + g) % nb

            @pl.when(k + g < cpw)
            def _():
                @pl.when(k >= nb - g)
                def _():
                    swait(b2)

                gather(k + g, b2)

        def loop_body(i, carry):
            for b in range(nb):
                chunk_step(i * nb + b, b)
            return carry

        nloop = cpw // nb
        lax.fori_loop(0, nloop, loop_body, 0)
        for k in range(nloop * nb, cpw):
            chunk_step(k, k % nb)
        for b in range(nb):
            swait(b)

        plsc.subcore_barrier()
        pltpu.sync_copy(acc.at[pl.ds(r0, rpt)], out_hbm.at[c, pl.ds(r0, rpt)])

    kern = pl.kernel(
        body,
        out_type=jax.ShapeDtypeStruct((_NC, n, f), jnp.float32),
        mesh=mesh,
        scratch_types=[
            pltpu.VMEM_SHARED((n, f), jnp.float32),
            pltpu.VMEM((epw,), jnp.int32),
            pltpu.VMEM((epw,), jnp.int32),
            [pltpu.VMEM((w, f), jnp.float32) for _ in range(nb)],
            [pltpu.SemaphoreType.DMA for _ in range(nb)],
            [pltpu.SemaphoreType.DMA for _ in range(nb)],
        ],
        compiler_params=pltpu.CompilerParams(use_tc_tiling_on_sc=False),
    )
    return kern(feat, ei, init)


def kernel(x, edge_index, W1l, b1, W1r, W2l, b2, W2r):
    n, d = x.shape
    h = W1l.shape[0]
    c = W2l.shape[0]

    xl, xr = _mm_split(x, W1l, W1r, b1[None, :])

    aggs1 = _seg_sum_sc(xl, edge_index, xr, 0, nb=6, g=5, w=40)

    y2 = _layer2(aggs1, W2l, W2r, b2[None, :])

    aggs2 = _seg_sum_sc(y2[:, :c], edge_index, y2, c, nb=6, g=5, w=_W)

    out_p = _combine(aggs2.reshape(2, n // 2, 2 * c))
    return out_p.reshape(n, c)
